# baseline (speedup 1.0000x reference)
import functools

import jax
import jax.numpy as jnp
from jax import lax
from jax.experimental import pallas as pl
from jax.experimental.pallas import tpu as pltpu
from jax.experimental.pallas import tpu_sc as plsc

D_MODEL = 1024
SCALE = float(D_MODEL) ** 0.5

_NUM_WORKERS = 32
_LANES = 16
_NG = 4
_NS = 2
_IDX_HEAD = 128


@functools.cache
def _make_emb_kernel(batch, seq, d_model, chunk):
    n_tokens = batch * seq
    b_per_w = n_tokens // _NUM_WORKERS
    w_per_row = seq // b_per_w
    n_chunks = b_per_w // chunk
    slices_per_row = d_model // _LANES
    mesh = plsc.VectorSubcoreMesh(core_axis_name="c", subcore_axis_name="s")

    steady_len = ((n_chunks - (_NG - 1) - _NS) // _NG) * _NG
    steady_end = _NS + steady_len

    @functools.partial(
        pl.kernel,
        out_type=jax.ShapeDtypeStruct((batch, seq, d_model), jnp.float32),
        mesh=mesh,
        scratch_types=[
            pltpu.VMEM((b_per_w,), jnp.int32),
            [pltpu.VMEM((chunk, d_model), jnp.float32) for _ in range(_NG)],
            [pltpu.VMEM((chunk, d_model), jnp.float32) for _ in range(_NS)],
            [pltpu.SemaphoreType.DMA for _ in range(_NG)],
            [pltpu.SemaphoreType.DMA for _ in range(_NS)],
            pltpu.SemaphoreType.DMA,
        ],
    )
    def emb(x_hbm, table_hbm, out_hbm, idx_v, gbuf, sbuf, gsem, ssem, isem):
        wid = lax.axis_index("s") * 2 + lax.axis_index("c")
        row = wid // w_per_row
        col0 = (wid % w_per_row) * b_per_w

        pltpu.sync_copy(x_hbm.at[row, pl.ds(col0, _IDX_HEAD)],
                        idx_v.at[pl.ds(0, _IDX_HEAD)])
        idx_rest = pltpu.make_async_copy(
            x_hbm.at[row, pl.ds(col0 + _IDX_HEAD, b_per_w - _IDX_HEAD)],
            idx_v.at[pl.ds(_IDX_HEAD, b_per_w - _IDX_HEAD)],
            isem,
        )
        idx_rest.start()

        def issue_gather(k, bg):
            pltpu.make_async_copy(
                table_hbm.at[idx_v.at[pl.ds(k * chunk, chunk)]],
                gbuf[bg], gsem[bg],
            ).start()

        def wait_gather(bg):
            pltpu.make_async_copy(
                table_hbm.at[pl.ds(0, chunk)], gbuf[bg], gsem[bg]
            ).wait()

        def issue_store(k, bs):
            pltpu.make_async_copy(
                sbuf[bs],
                out_hbm.at[row, pl.ds(col0 + k * chunk, chunk), :],
                ssem[bs],
            ).start()

        def wait_store(bs):
            pltpu.make_async_copy(
                sbuf[bs], out_hbm.at[0, pl.ds(0, chunk), :], ssem[bs]
            ).wait()

        def scale_chunk(bg, bs):
            def row_body(r, _):
                for j in range(slices_per_row):
                    sl = pl.ds(j * _LANES, _LANES)
                    sbuf[bs][r, sl] = gbuf[bg][r, sl] * SCALE
                return 0

            lax.fori_loop(0, chunk, row_body, 0)

        def process(k, kmod, wait_s, do_gather):
            if do_gather:
                issue_gather(k + _NG - 1, (kmod + _NG - 1) % _NG)
            wait_gather(kmod % _NG)
            if wait_s:
                wait_store(kmod % _NS)
            scale_chunk(kmod % _NG, kmod % _NS)
            issue_store(k, kmod % _NS)

        for k in range(_NG - 1):
            issue_gather(k, k)

        for k in range(_NS):
            process(k, k, wait_s=False, do_gather=True)

        idx_rest.wait()

        def group_body(kg, _):
            k0 = kg * _NG + _NS
            for i in range(_NG):
                process(k0 + i, _NS + i, wait_s=True, do_gather=True)
            return 0

        lax.fori_loop(0, steady_len // _NG, group_body, 0)

        for k in range(steady_end, n_chunks):
            process(k, k, wait_s=True, do_gather=(k + _NG - 1 < n_chunks))
        for k in range(n_chunks - _NS, n_chunks):
            wait_store(k % _NS)

    return emb


@jax.jit
def kernel(x, table):
    batch, seq = x.shape
    return _make_emb_kernel(batch, seq, D_MODEL, 16)(
        x.astype(jnp.int32), table)


# speedup vs baseline: 1.0020x; 1.0020x over previous
import functools

import jax
import jax.numpy as jnp
from jax import lax
from jax.experimental import pallas as pl
from jax.experimental.pallas import tpu as pltpu
from jax.experimental.pallas import tpu_sc as plsc

D_MODEL = 1024
SCALE = float(D_MODEL) ** 0.5

_NUM_WORKERS = 32
_LANES = 16
_NG = 4
_NS = 2
_IDX_HEAD = 128


@functools.cache
def _make_emb_kernel(n_tokens, d_model, chunk):
    b_per_w = n_tokens // _NUM_WORKERS
    n_chunks = b_per_w // chunk
    slices_per_row = d_model // _LANES
    mesh = plsc.VectorSubcoreMesh(core_axis_name="c", subcore_axis_name="s")

    steady_len = ((n_chunks - (_NG - 1) - _NS) // _NG) * _NG
    steady_end = _NS + steady_len

    @functools.partial(
        pl.kernel,
        out_type=jax.ShapeDtypeStruct((n_tokens, d_model), jnp.float32),
        mesh=mesh,
        scratch_types=[
            pltpu.VMEM((b_per_w,), jnp.int32),
            [pltpu.VMEM((chunk, d_model), jnp.float32) for _ in range(_NG)],
            [pltpu.VMEM((chunk, d_model), jnp.float32) for _ in range(_NS)],
            [pltpu.SemaphoreType.DMA for _ in range(_NG)],
            [pltpu.SemaphoreType.DMA for _ in range(_NS)],
            pltpu.SemaphoreType.DMA,
        ],
    )
    def emb(x_hbm, table_hbm, out_hbm, idx_v, gbuf, sbuf, gsem, ssem, isem):
        wid = lax.axis_index("s") * 2 + lax.axis_index("c")
        base = wid * b_per_w

        pltpu.sync_copy(x_hbm.at[pl.ds(base, _IDX_HEAD)],
                        idx_v.at[pl.ds(0, _IDX_HEAD)])
        idx_rest = pltpu.make_async_copy(
            x_hbm.at[pl.ds(base + _IDX_HEAD, b_per_w - _IDX_HEAD)],
            idx_v.at[pl.ds(_IDX_HEAD, b_per_w - _IDX_HEAD)],
            isem,
        )
        idx_rest.start()

        def issue_gather(k, bg):
            pltpu.make_async_copy(
                table_hbm.at[idx_v.at[pl.ds(k * chunk, chunk)]],
                gbuf[bg], gsem[bg],
            ).start()

        def wait_gather(bg):
            pltpu.make_async_copy(
                table_hbm.at[pl.ds(0, chunk)], gbuf[bg], gsem[bg]
            ).wait()

        def issue_store(k, bs):
            pltpu.make_async_copy(
                sbuf[bs], out_hbm.at[pl.ds(base + k * chunk, chunk)], ssem[bs]
            ).start()

        def wait_store(bs):
            pltpu.make_async_copy(
                sbuf[bs], out_hbm.at[pl.ds(0, chunk)], ssem[bs]
            ).wait()

        def scale_chunk(bg, bs):
            def row_body(r, _):
                for j in range(slices_per_row):
                    sl = pl.ds(j * _LANES, _LANES)
                    sbuf[bs][r, sl] = gbuf[bg][r, sl] * SCALE
                return 0

            lax.fori_loop(0, chunk, row_body, 0)

        def process(k, kmod, wait_s, do_gather):
            if do_gather:
                issue_gather(k + _NG - 1, (kmod + _NG - 1) % _NG)
            wait_gather(kmod % _NG)
            if wait_s:
                wait_store(kmod % _NS)
            scale_chunk(kmod % _NG, kmod % _NS)
            issue_store(k, kmod % _NS)

        for k in range(_NG - 1):
            issue_gather(k, k)

        for k in range(_NS):
            process(k, k, wait_s=False, do_gather=True)

        idx_rest.wait()

        def group_body(kg, _):
            k0 = kg * _NG + _NS
            for i in range(_NG):
                process(k0 + i, _NS + i, wait_s=True, do_gather=True)
            return 0

        lax.fori_loop(0, steady_len // _NG, group_body, 0)

        for k in range(steady_end, n_chunks):
            process(k, k, wait_s=True, do_gather=(k + _NG - 1 < n_chunks))
        for k in range(n_chunks - _NS, n_chunks):
            wait_store(k % _NS)

    return emb


@jax.jit
def kernel(x, table):
    batch, seq = x.shape
    x_flat = x.reshape(batch * seq).astype(jnp.int32)
    out = _make_emb_kernel(batch * seq, D_MODEL, 16)(x_flat, table)
    return out.reshape(batch, seq, D_MODEL)


# speedup vs baseline: 1.0114x; 1.0094x over previous
import functools

import jax
import jax.numpy as jnp
from jax import lax
from jax.experimental import pallas as pl
from jax.experimental.pallas import tpu as pltpu
from jax.experimental.pallas import tpu_sc as plsc

D_MODEL = 1024
SCALE = float(D_MODEL) ** 0.5

_NUM_WORKERS = 32
_LANES = 16
_NBUF = 3
_IDX_HEAD = 128


@functools.cache
def _make_emb_kernel(n_tokens, d_model, chunk):
    b_per_w = n_tokens // _NUM_WORKERS
    n_chunks = b_per_w // chunk
    slices_per_row = d_model // _LANES
    mesh = plsc.VectorSubcoreMesh(core_axis_name="c", subcore_axis_name="s")

    steady_len = ((n_chunks - 2 * _NBUF) // _NBUF) * _NBUF
    steady_end = _NBUF + steady_len

    @functools.partial(
        pl.kernel,
        out_type=jax.ShapeDtypeStruct((n_tokens, d_model), jnp.float32),
        mesh=mesh,
        scratch_types=[
            pltpu.VMEM((b_per_w,), jnp.int32),
            [pltpu.VMEM((chunk, d_model), jnp.float32) for _ in range(_NBUF)],
            [pltpu.VMEM((chunk, d_model), jnp.float32) for _ in range(_NBUF)],
            [pltpu.SemaphoreType.DMA for _ in range(_NBUF)],
            [pltpu.SemaphoreType.DMA for _ in range(_NBUF)],
            pltpu.SemaphoreType.DMA,
        ],
    )
    def emb(x_hbm, table_hbm, out_hbm, idx_v, gbuf, sbuf, gsem, ssem, isem):
        wid = lax.axis_index("s") * 2 + lax.axis_index("c")
        base = wid * b_per_w

        pltpu.sync_copy(x_hbm.at[pl.ds(base, _IDX_HEAD)],
                        idx_v.at[pl.ds(0, _IDX_HEAD)])
        idx_rest = pltpu.make_async_copy(
            x_hbm.at[pl.ds(base + _IDX_HEAD, b_per_w - _IDX_HEAD)],
            idx_v.at[pl.ds(_IDX_HEAD, b_per_w - _IDX_HEAD)],
            isem,
        )
        idx_rest.start()

        def issue_gather(k, b):
            pltpu.make_async_copy(
                table_hbm.at[idx_v.at[pl.ds(k * chunk, chunk)]],
                gbuf[b], gsem[b],
            ).start()

        def wait_gather(b):
            pltpu.make_async_copy(
                table_hbm.at[pl.ds(0, chunk)], gbuf[b], gsem[b]
            ).wait()

        def issue_store(k, b):
            pltpu.make_async_copy(
                sbuf[b], out_hbm.at[pl.ds(base + k * chunk, chunk)], ssem[b]
            ).start()

        def wait_store(b):
            pltpu.make_async_copy(
                sbuf[b], out_hbm.at[pl.ds(0, chunk)], ssem[b]
            ).wait()

        def scale_chunk(b):
            def row_body(r, _):
                for j in range(slices_per_row):
                    sl = pl.ds(j * _LANES, _LANES)
                    sbuf[b][r, sl] = gbuf[b][r, sl] * SCALE
                return 0

            lax.fori_loop(0, chunk, row_body, 0)

        def process(k, b, wait_s, do_gather):
            wait_gather(b)
            if wait_s:
                wait_store(b)
            scale_chunk(b)
            if do_gather:
                issue_gather(k + _NBUF, b)
            issue_store(k, b)

        for b in range(_NBUF):
            issue_gather(b, b)
        for b in range(_NBUF):
            process(b, b, wait_s=False, do_gather=True)

        idx_rest.wait()

        def group_body(kg, _):
            k0 = kg * _NBUF
            for b in range(_NBUF):
                process(k0 + b, b, wait_s=True, do_gather=True)
            return 0

        lax.fori_loop(1, steady_end // _NBUF, group_body, 0)

        for k in range(steady_end, n_chunks):
            process(k, k % _NBUF, wait_s=True,
                    do_gather=(k + _NBUF < n_chunks))
        for k in range(n_chunks - _NBUF, n_chunks):
            wait_store(k % _NBUF)

    return emb


@jax.jit
def kernel(x, table):
    batch, seq = x.shape
    x_flat = x.reshape(batch * seq).astype(jnp.int32)
    out = _make_emb_kernel(batch * seq, D_MODEL, 16)(x_flat, table)
    return out.reshape(batch, seq, D_MODEL)
